# baseline (speedup 1.0000x reference)
import functools

import jax
import jax.numpy as jnp
from jax import lax
from jax.experimental import pallas as pl
from jax.experimental.pallas import tpu as pltpu
from jax.experimental.pallas import tpu_sc as plsc

N = 10000
E = 320000
D = 128
A = 64

NCORES = 2
NSUB = 16
NTILES = NCORES * NSUB
E2 = E // 2
EPT = E2 // NTILES
CH = 40
NCHUNK = EPT // CH
NPAD = 10240
RPT = NPAD // NSUB
ZR = 64

_MESH = plsc.VectorSubcoreMesh(core_axis_name="c", subcore_axis_name="s")


def _node_body(x_ref, wqk_ref, wy_ref, by_ref, qk_ref, vy_ref):
    x = x_ref[...]
    qk_ref[...] = jnp.dot(x, wqk_ref[...], preferred_element_type=jnp.float32)
    vy_ref[...] = (
        jnp.dot(x, wy_ref[...], preferred_element_type=jnp.float32) + by_ref[...]
    )


def _node_stage(X, Wqk, Wy, by):
    return pl.pallas_call(
        _node_body,
        out_shape=[
            jax.ShapeDtypeStruct((N, D), jnp.float32),
            jax.ShapeDtypeStruct((N, D), jnp.float32),
        ],
    )(X, Wqk, Wy, by.reshape(1, D))


def _gather_body(qk_hbm, src_hbm, dst_hbm, g_hbm,
                 sidx, didx, qks, qkd, gb,
                 sem_si, sem_di, sem_gq, sem_gk, sem_wb):
    c = lax.axis_index("c")
    s = lax.axis_index("s")
    tile_base = (c * NSUB + s) * EPT

    def start_idx(i, b):
        base = tile_base + i * CH
        pltpu.async_copy(src_hbm.at[pl.ds(base, CH)], sidx.at[b], sem_si)
        pltpu.async_copy(dst_hbm.at[pl.ds(base, CH)], didx.at[b], sem_di)

    start_idx(0, 0)

    def chunk(i, carry):
        b = lax.rem(i, 2)
        nb = 1 - b
        pltpu.make_async_copy(src_hbm.at[pl.ds(0, CH)], sidx.at[b], sem_si).wait()
        pltpu.make_async_copy(dst_hbm.at[pl.ds(0, CH)], didx.at[b], sem_di).wait()
        pltpu.async_copy(qk_hbm.at[sidx.at[b]], qks.at[b], sem_gq)
        pltpu.async_copy(qk_hbm.at[didx.at[b]], qkd.at[b], sem_gk)

        @pl.when(i >= 1)
        def _():
            pltpu.make_async_copy(
                gb.at[nb], g_hbm.at[pl.ds(0, CH)], sem_wb
            ).wait()

        @pl.when(i + 1 < NCHUNK)
        def _():
            start_idx(i + 1, nb)

        pltpu.make_async_copy(qk_hbm.at[sidx.at[b]], qks.at[b], sem_gq).wait()
        pltpu.make_async_copy(qk_hbm.at[didx.at[b]], qkd.at[b], sem_gk).wait()

        @plsc.parallel_loop(0, CH, unroll=8)
        def addrow(r):
            for l in range(A // 16):
                gb[b, r, pl.ds(l * 16, 16)] = (
                    qks[b, r, pl.ds(l * 16, 16)]
                    + qkd[b, r, pl.ds(A + l * 16, 16)]
                )

        base = tile_base + i * CH
        pltpu.async_copy(gb.at[b], g_hbm.at[pl.ds(base, CH)], sem_wb)
        return carry

    lax.fori_loop(0, NCHUNK, chunk, 0)
    pltpu.make_async_copy(gb.at[0], g_hbm.at[pl.ds(0, CH)], sem_wb).wait()


def _gather_stage(qk, src_h, dst_h):
    f = functools.partial(
        pl.kernel,
        out_type=jax.ShapeDtypeStruct((E2, A), jnp.float32),
        mesh=_MESH,
        scratch_types=[
            pltpu.VMEM((2, CH), jnp.int32),
            pltpu.VMEM((2, CH), jnp.int32),
            pltpu.VMEM((2, CH, D), jnp.float32),
            pltpu.VMEM((2, CH, D), jnp.float32),
            pltpu.VMEM((2, CH, A), jnp.float32),
            pltpu.SemaphoreType.DMA,
            pltpu.SemaphoreType.DMA,
            pltpu.SemaphoreType.DMA,
            pltpu.SemaphoreType.DMA,
            pltpu.SemaphoreType.DMA,
        ],
    )(_gather_body)
    return f(qk, src_h, dst_h)


BE = 2000


def _edge_body(g_ref, wphi_ref, we_ref, be_ref, wve_ref, wb_ref):
    e = jnp.tanh(g_ref[...])
    sc = jnp.sum(e * wphi_ref[...], axis=1, keepdims=True)
    w = jnp.exp(sc)
    ve = jnp.dot(e, we_ref[...], preferred_element_type=jnp.float32) + be_ref[...]
    wve_ref[...] = w * ve
    wb_ref[...] = jnp.broadcast_to(w, (BE, 16))


def _edge_stage(g, Wphi, We, be):
    return pl.pallas_call(
        _edge_body,
        grid=(E2 // BE,),
        in_specs=[
            pl.BlockSpec((BE, A), lambda i: (i, 0)),
            pl.BlockSpec((1, A), lambda i: (0, 0)),
            pl.BlockSpec((A, D), lambda i: (0, 0)),
            pl.BlockSpec((1, D), lambda i: (0, 0)),
        ],
        out_specs=[
            pl.BlockSpec((BE, D), lambda i: (i, 0)),
            pl.BlockSpec((BE, 16), lambda i: (i, 0)),
        ],
        out_shape=[
            jax.ShapeDtypeStruct((E2, D), jnp.float32),
            jax.ShapeDtypeStruct((E2, 16), jnp.float32),
        ],
    )(g, Wphi.reshape(1, A), We, be.reshape(1, D))


def _scatter_body(wve0_hbm, wbb0_hbm, wve1_hbm, wbb1_hbm, vy_hbm,
                  src0_hbm, dst0_hbm, src1_hbm, dst1_hbm, z_hbm,
                  p0_hbm, p1_hbm, pd0_hbm, pd1_hbm,
                  sidx, didx, mb, vyb, wbb,
                  sem_si, sem_di, sem_m, sem_w, sem_v, sem_sc, acc_sh):
    c = lax.axis_index("c")
    s = lax.axis_index("s")
    tile = c * NSUB + s
    tile_base = tile * EPT

    def zcopy(j, carry):
        pltpu.sync_copy(z_hbm, acc_sh.at[pl.ds(s * RPT + j * ZR, ZR)])
        return carry

    lax.fori_loop(0, RPT // ZR, zcopy, 0)
    plsc.subcore_barrier()

    def wait_sc():
        pltpu.make_async_copy(mb.at[0], acc_sh.at[sidx.at[0]], sem_sc).wait()

    def run_phase1(wve_hbm, wbb_hbm, src_hbm, dst_hbm):
        def start_loads(i, b):
            base = tile_base + i * CH
            pltpu.async_copy(src_hbm.at[pl.ds(base, CH)], sidx.at[b], sem_si)
            pltpu.async_copy(dst_hbm.at[pl.ds(base, CH)], didx.at[b], sem_di)
            pltpu.async_copy(wve_hbm.at[pl.ds(base, CH)], mb.at[b], sem_m)
            pltpu.async_copy(
                wbb_hbm.at[pl.ds(base * 16, CH * 16)], wbb.at[b], sem_w
            )

        start_loads(0, 0)

        def chunk(i, carry):
            b = lax.rem(i, 2)
            nb = 1 - b
            pltpu.make_async_copy(
                src_hbm.at[pl.ds(0, CH)], sidx.at[b], sem_si
            ).wait()
            pltpu.make_async_copy(
                dst_hbm.at[pl.ds(0, CH)], didx.at[b], sem_di
            ).wait()
            cv = pltpu.async_copy(vy_hbm.at[didx.at[b]], vyb.at[b], sem_v)

            @pl.when(i >= 1)
            def _():
                wait_sc()

            @pl.when(i + 1 < NCHUNK)
            def _():
                start_loads(i + 1, nb)

            pltpu.make_async_copy(
                wve_hbm.at[pl.ds(0, CH)], mb.at[b], sem_m
            ).wait()
            pltpu.make_async_copy(
                wbb_hbm.at[pl.ds(0, CH * 16)], wbb.at[b], sem_w
            ).wait()
            cv.wait()

            @plsc.parallel_loop(0, CH, unroll=8)
            def frow(r):
                wv = wbb[b, pl.ds(r * 16, 16)]
                for l in range(D // 16):
                    sl = pl.ds(l * 16, 16)
                    mb[b, r, sl] = mb[b, r, sl] + wv * vyb[b, r, sl]

            pltpu.async_copy(mb.at[b], acc_sh.at[sidx.at[b]], sem_sc, add=True)
            return carry

        lax.fori_loop(0, NCHUNK, chunk, 0)
        wait_sc()

    def run_phase2(wbb_hbm, src_hbm):
        def start_dloads(i, b):
            base = tile_base + i * CH
            pltpu.async_copy(src_hbm.at[pl.ds(base, CH)], sidx.at[b], sem_si)
            pltpu.async_copy(
                wbb_hbm.at[pl.ds(base * 16, CH * 16)], wbb.at[b], sem_w
            )

        start_dloads(0, 0)

        def dchunk(i, carry):
            b = lax.rem(i, 2)
            nb = 1 - b
            pltpu.make_async_copy(
                src_hbm.at[pl.ds(0, CH)], sidx.at[b], sem_si
            ).wait()

            @pl.when(i >= 1)
            def _():
                wait_sc()

            @pl.when(i + 1 < NCHUNK)
            def _():
                start_dloads(i + 1, nb)

            pltpu.make_async_copy(
                wbb_hbm.at[pl.ds(0, CH * 16)], wbb.at[b], sem_w
            ).wait()

            @plsc.parallel_loop(0, CH, unroll=8)
            def wrow(r):
                wv = wbb[b, pl.ds(r * 16, 16)]
                mb[b, r, pl.ds(0, 16)] = wv

            pltpu.async_copy(mb.at[b], acc_sh.at[sidx.at[b]], sem_sc, add=True)
            return carry

        lax.fori_loop(0, NCHUNK, dchunk, 0)
        wait_sc()

    run_phase1(wve0_hbm, wbb0_hbm, src0_hbm, dst0_hbm)
    run_phase1(wve1_hbm, wbb1_hbm, src1_hbm, dst1_hbm)
    plsc.subcore_barrier()

    row0 = s * RPT

    @pl.when(c == 0)
    def _():
        pltpu.sync_copy(acc_sh.at[pl.ds(row0, RPT)], p0_hbm.at[pl.ds(row0, RPT)])

    @pl.when(c == 1)
    def _():
        pltpu.sync_copy(acc_sh.at[pl.ds(row0, RPT)], p1_hbm.at[pl.ds(row0, RPT)])

    plsc.subcore_barrier()
    lax.fori_loop(0, RPT // ZR, zcopy, 0)
    plsc.subcore_barrier()

    run_phase2(wbb0_hbm, src0_hbm)
    run_phase2(wbb1_hbm, src1_hbm)
    plsc.subcore_barrier()

    @pl.when(c == 0)
    def _():
        pltpu.sync_copy(acc_sh.at[pl.ds(row0, RPT)], pd0_hbm.at[pl.ds(row0, RPT)])

    @pl.when(c == 1)
    def _():
        pltpu.sync_copy(acc_sh.at[pl.ds(row0, RPT)], pd1_hbm.at[pl.ds(row0, RPT)])


def _scatter_stage(wve0, wbb0, wve1, wbb1, vy, src0, dst0, src1, dst1):
    f = functools.partial(
        pl.kernel,
        out_type=[
            jax.ShapeDtypeStruct((NPAD, D), jnp.float32),
            jax.ShapeDtypeStruct((NPAD, D), jnp.float32),
            jax.ShapeDtypeStruct((NPAD, D), jnp.float32),
            jax.ShapeDtypeStruct((NPAD, D), jnp.float32),
        ],
        mesh=_MESH,
        scratch_types=[
            pltpu.VMEM((2, CH), jnp.int32),
            pltpu.VMEM((2, CH), jnp.int32),
            pltpu.VMEM((2, CH, D), jnp.float32),
            pltpu.VMEM((2, CH, D), jnp.float32),
            pltpu.VMEM((2, CH * 16), jnp.float32),
            pltpu.SemaphoreType.DMA,
            pltpu.SemaphoreType.DMA,
            pltpu.SemaphoreType.DMA,
            pltpu.SemaphoreType.DMA,
            pltpu.SemaphoreType.DMA,
            pltpu.SemaphoreType.DMA,
            pltpu.VMEM_SHARED((NPAD, D), jnp.float32),
        ],
    )(_scatter_body)
    return f(wve0, wbb0.reshape(E2 * 16), wve1, wbb1.reshape(E2 * 16), vy,
             src0, dst0, src1, dst1, jnp.zeros((ZR, D), jnp.float32))


RB = 2000


def _final_body(p0_ref, p1_ref, pd0_ref, pd1_ref, o_ref):
    psum = p0_ref[...] + p1_ref[...]
    den = pd0_ref[...][:, 0:1] + pd1_ref[...][:, 0:1]
    o_ref[...] = psum / (den + 1e-9)


def _final_stage(p0, p1, pd0, pd1):
    return pl.pallas_call(
        _final_body,
        grid=(N // RB,),
        in_specs=[
            pl.BlockSpec((RB, D), lambda i: (i, 0)),
            pl.BlockSpec((RB, D), lambda i: (i, 0)),
            pl.BlockSpec((RB, D), lambda i: (i, 0)),
            pl.BlockSpec((RB, D), lambda i: (i, 0)),
        ],
        out_specs=pl.BlockSpec((RB, D), lambda i: (i, 0)),
        out_shape=jax.ShapeDtypeStruct((N, D), jnp.float32),
    )(p0, p1, pd0, pd1)


def kernel(X, edge_index, Wq, Wk, Wphi, Wy, by, We, be):
    src0 = edge_index[0, :E2]
    dst0 = edge_index[1, :E2]
    src1 = edge_index[0, E2:]
    dst1 = edge_index[1, E2:]
    Wqk = jnp.concatenate([Wq, Wk], axis=1)
    qk, vy = _node_stage(X, Wqk, Wy, by)
    g0 = _gather_stage(qk, src0, dst0)
    g1 = _gather_stage(qk, src1, dst1)
    wve0, wbb0 = _edge_stage(g0, Wphi, We, be)
    wve1, wbb1 = _edge_stage(g1, Wphi, We, be)
    p0, p1, pd0, pd1 = _scatter_stage(
        wve0, wbb0, wve1, wbb1, vy, src0, dst0, src1, dst1
    )
    return _final_stage(p0, p1, pd0, pd1)


# speedup vs baseline: 1.1378x; 1.1378x over previous
import functools

import jax
import jax.numpy as jnp
from jax import lax
from jax.experimental import pallas as pl
from jax.experimental.pallas import tpu as pltpu
from jax.experimental.pallas import tpu_sc as plsc

N = 10000
E = 320000
D = 128
A = 64

NCORES = 2
NSUB = 16
NTILES = NCORES * NSUB
EPT = E // NTILES
CH = 80
NCHUNK = EPT // CH
NPAD = 10240
RPT = NPAD // NSUB
ZR = 64

_MESH = plsc.VectorSubcoreMesh(core_axis_name="c", subcore_axis_name="s")


def _zv():
    return jnp.zeros((16,), jnp.float32)


def _node_body(x_ref, wqk_ref, wy_ref, by_ref, qk_ref, vy_ref):
    x = x_ref[...]
    qk_ref[...] = jnp.dot(x, wqk_ref[...], preferred_element_type=jnp.float32)
    vy_ref[...] = (
        jnp.dot(x, wy_ref[...], preferred_element_type=jnp.float32) + by_ref[...]
    )


def _node_stage(X, Wqk, Wy, by):
    return pl.pallas_call(
        _node_body,
        out_shape=[
            jax.ShapeDtypeStruct((N, D), jnp.float32),
            jax.ShapeDtypeStruct((N, D), jnp.float32),
        ],
    )(X, Wqk, Wy, by.reshape(1, D))


def _gather_body(qk_hbm, src_hbm, dst_hbm, g_hbm,
                 sidx, didx, qks, qkd, gb,
                 sem_si, sem_di, sem_gq, sem_gk, sem_wb):
    c = lax.axis_index("c")
    s = lax.axis_index("s")
    tile_base = (c * NSUB + s) * EPT

    def start_idx(i, b):
        base = tile_base + i * CH
        pltpu.async_copy(src_hbm.at[pl.ds(base, CH)], sidx.at[b], sem_si)
        pltpu.async_copy(dst_hbm.at[pl.ds(base, CH)], didx.at[b], sem_di)

    start_idx(0, 0)

    def chunk(i, carry):
        b = lax.rem(i, 2)
        nb = 1 - b
        pltpu.make_async_copy(src_hbm.at[pl.ds(0, CH)], sidx.at[b], sem_si).wait()
        pltpu.make_async_copy(dst_hbm.at[pl.ds(0, CH)], didx.at[b], sem_di).wait()
        pltpu.async_copy(qk_hbm.at[sidx.at[b]], qks.at[b], sem_gq)
        pltpu.async_copy(qk_hbm.at[didx.at[b]], qkd.at[b], sem_gk)

        @pl.when(i >= 1)
        def _():
            pltpu.make_async_copy(
                gb.at[nb], g_hbm.at[pl.ds(0, CH)], sem_wb
            ).wait()

        @pl.when(i + 1 < NCHUNK)
        def _():
            start_idx(i + 1, nb)

        pltpu.make_async_copy(qk_hbm.at[sidx.at[b]], qks.at[b], sem_gq).wait()
        pltpu.make_async_copy(qk_hbm.at[didx.at[b]], qkd.at[b], sem_gk).wait()

        @plsc.parallel_loop(0, CH, unroll=8)
        def addrow(r):
            for l in range(A // 16):
                gb[b, r, pl.ds(l * 16, 16)] = (
                    qks[b, r, pl.ds(l * 16, 16)]
                    + qkd[b, r, pl.ds(A + l * 16, 16)]
                )
        base = tile_base + i * CH
        pltpu.async_copy(gb.at[b], g_hbm.at[pl.ds(base, CH)], sem_wb)
        return carry

    lax.fori_loop(0, NCHUNK, chunk, 0)
    pltpu.make_async_copy(gb.at[0], g_hbm.at[pl.ds(0, CH)], sem_wb).wait()


def _gather_stage(qk, src, dst):
    f = functools.partial(
        pl.kernel,
        out_type=jax.ShapeDtypeStruct((E, A), jnp.float32),
        mesh=_MESH,
        scratch_types=[
            pltpu.VMEM((2, CH), jnp.int32),
            pltpu.VMEM((2, CH), jnp.int32),
            pltpu.VMEM((2, CH, D), jnp.float32),
            pltpu.VMEM((2, CH, D), jnp.float32),
            pltpu.VMEM((2, CH, A), jnp.float32),
            pltpu.SemaphoreType.DMA,
            pltpu.SemaphoreType.DMA,
            pltpu.SemaphoreType.DMA,
            pltpu.SemaphoreType.DMA,
            pltpu.SemaphoreType.DMA,
        ],
    )(_gather_body)
    return f(qk, src, dst)


BE = 2560


def _edge_body(g_ref, wphi_ref, we_ref, be_ref, wve_ref, wb_ref):
    e = jnp.tanh(g_ref[...])
    sc = jnp.sum(e * wphi_ref[...], axis=1, keepdims=True)
    w = jnp.exp(sc)
    ve = jnp.dot(
        e.astype(jnp.bfloat16),
        we_ref[...].astype(jnp.bfloat16),
        preferred_element_type=jnp.float32,
    ) + be_ref[...]
    wve_ref[...] = w * ve
    wb_ref[...] = jnp.broadcast_to(w, (BE, 16))


def _edge_stage(g, Wphi, We, be):
    return pl.pallas_call(
        _edge_body,
        grid=(E // BE,),
        in_specs=[
            pl.BlockSpec((BE, A), lambda i: (i, 0)),
            pl.BlockSpec((1, A), lambda i: (0, 0)),
            pl.BlockSpec((A, D), lambda i: (0, 0)),
            pl.BlockSpec((1, D), lambda i: (0, 0)),
        ],
        out_specs=[
            pl.BlockSpec((BE, D), lambda i: (i, 0)),
            pl.BlockSpec((BE, 16), lambda i: (i, 0)),
        ],
        out_shape=[
            jax.ShapeDtypeStruct((E, D), jnp.float32),
            jax.ShapeDtypeStruct((E, 16), jnp.float32),
        ],
    )(g, Wphi.reshape(1, A), We, be.reshape(1, D))


def _scatter_body(wve_hbm, wbb_hbm, vy_hbm, src_hbm, dst_hbm, z_hbm,
                  p0_hbm, p1_hbm, pd0_hbm, pd1_hbm,
                  sidx, didx, mb, vyb, wbb,
                  sem_si, sem_di, sem_m, sem_w, sem_v, sem_sc, acc_sh):
    c = lax.axis_index("c")
    s = lax.axis_index("s")
    tile = c * NSUB + s
    tile_base = tile * EPT

    def zcopy(j, carry):
        pltpu.sync_copy(z_hbm, acc_sh.at[pl.ds(s * RPT + j * ZR, ZR)])
        return carry

    lax.fori_loop(0, RPT // ZR, zcopy, 0)
    plsc.subcore_barrier()

    def start_loads(i, b):
        base = tile_base + i * CH
        pltpu.async_copy(src_hbm.at[pl.ds(base, CH)], sidx.at[b], sem_si)
        pltpu.async_copy(dst_hbm.at[pl.ds(base, CH)], didx.at[b], sem_di)
        pltpu.async_copy(wve_hbm.at[pl.ds(base, CH)], mb.at[b], sem_m)
        pltpu.async_copy(wbb_hbm.at[pl.ds(base * 16, CH * 16)], wbb.at[b], sem_w)

    def wait_sc():
        pltpu.make_async_copy(mb.at[0], acc_sh.at[sidx.at[0]], sem_sc).wait()

    start_loads(0, 0)

    def chunk(i, carry):
        b = lax.rem(i, 2)
        nb = 1 - b
        pltpu.make_async_copy(src_hbm.at[pl.ds(0, CH)], sidx.at[b], sem_si).wait()
        pltpu.make_async_copy(dst_hbm.at[pl.ds(0, CH)], didx.at[b], sem_di).wait()
        cv = pltpu.async_copy(vy_hbm.at[didx.at[b]], vyb.at[b], sem_v)

        @pl.when(i >= 1)
        def _():
            wait_sc()

        @pl.when(i + 1 < NCHUNK)
        def _():
            start_loads(i + 1, nb)

        pltpu.make_async_copy(wve_hbm.at[pl.ds(0, CH)], mb.at[b], sem_m).wait()
        pltpu.make_async_copy(wbb_hbm.at[pl.ds(0, CH * 16)], wbb.at[b], sem_w).wait()
        cv.wait()

        @plsc.parallel_loop(0, CH, unroll=8)
        def frow(r):
            wv = wbb[b, pl.ds(r * 16, 16)]
            for l in range(D // 16):
                sl = pl.ds(l * 16, 16)
                mb[b, r, sl] = mb[b, r, sl] + wv * vyb[b, r, sl]
        pltpu.async_copy(mb.at[b], acc_sh.at[sidx.at[b]], sem_sc, add=True)
        return carry

    lax.fori_loop(0, NCHUNK, chunk, 0)
    wait_sc()
    plsc.subcore_barrier()

    row0 = s * RPT

    @pl.when(c == 0)
    def _():
        pltpu.sync_copy(acc_sh.at[pl.ds(row0, RPT)], p0_hbm.at[pl.ds(row0, RPT)])

    @pl.when(c == 1)
    def _():
        pltpu.sync_copy(acc_sh.at[pl.ds(row0, RPT)], p1_hbm.at[pl.ds(row0, RPT)])

    plsc.subcore_barrier()
    lax.fori_loop(0, RPT // ZR, zcopy, 0)
    plsc.subcore_barrier()

    def start_dloads(i, b):
        base = tile_base + i * CH
        pltpu.async_copy(src_hbm.at[pl.ds(base, CH)], sidx.at[b], sem_si)
        pltpu.async_copy(wbb_hbm.at[pl.ds(base * 16, CH * 16)], wbb.at[b], sem_w)

    start_dloads(0, 0)

    def dchunk(i, carry):
        b = lax.rem(i, 2)
        nb = 1 - b
        pltpu.make_async_copy(src_hbm.at[pl.ds(0, CH)], sidx.at[b], sem_si).wait()

        @pl.when(i >= 1)
        def _():
            wait_sc()

        @pl.when(i + 1 < NCHUNK)
        def _():
            start_dloads(i + 1, nb)

        pltpu.make_async_copy(wbb_hbm.at[pl.ds(0, CH * 16)], wbb.at[b], sem_w).wait()

        @plsc.parallel_loop(0, CH, unroll=8)
        def wrow(r):
            wv = wbb[b, pl.ds(r * 16, 16)]
            mb[b, r, pl.ds(0, 16)] = wv
        pltpu.async_copy(mb.at[b], acc_sh.at[sidx.at[b]], sem_sc, add=True)
        return carry

    lax.fori_loop(0, NCHUNK, dchunk, 0)
    wait_sc()
    plsc.subcore_barrier()

    @pl.when(c == 0)
    def _():
        pltpu.sync_copy(acc_sh.at[pl.ds(row0, RPT)], pd0_hbm.at[pl.ds(row0, RPT)])

    @pl.when(c == 1)
    def _():
        pltpu.sync_copy(acc_sh.at[pl.ds(row0, RPT)], pd1_hbm.at[pl.ds(row0, RPT)])


def _scatter_stage(wve, wbb, vy, src, dst):
    f = functools.partial(
        pl.kernel,
        out_type=[
            jax.ShapeDtypeStruct((NPAD, D), jnp.float32),
            jax.ShapeDtypeStruct((NPAD, D), jnp.float32),
            jax.ShapeDtypeStruct((NPAD, D), jnp.float32),
            jax.ShapeDtypeStruct((NPAD, D), jnp.float32),
        ],
        mesh=_MESH,
        scratch_types=[
            pltpu.VMEM((2, CH), jnp.int32),
            pltpu.VMEM((2, CH), jnp.int32),
            pltpu.VMEM((2, CH, D), jnp.float32),
            pltpu.VMEM((2, CH, D), jnp.float32),
            pltpu.VMEM((2, CH * 16), jnp.float32),
            pltpu.SemaphoreType.DMA,
            pltpu.SemaphoreType.DMA,
            pltpu.SemaphoreType.DMA,
            pltpu.SemaphoreType.DMA,
            pltpu.SemaphoreType.DMA,
            pltpu.SemaphoreType.DMA,
            pltpu.VMEM_SHARED((NPAD, D), jnp.float32),
        ],
    )(_scatter_body)
    return f(wve, wbb.reshape(E * 16), vy, src, dst, jnp.zeros((ZR, D), jnp.float32))


RB = 2000


def _final_body(p0_ref, p1_ref, pd0_ref, pd1_ref, o_ref):
    psum = p0_ref[...] + p1_ref[...]
    den = pd0_ref[...][:, 0:1] + pd1_ref[...][:, 0:1]
    o_ref[...] = psum / (den + 1e-9)


def _final_stage(p0, p1, pd0, pd1):
    return pl.pallas_call(
        _final_body,
        grid=(N // RB,),
        in_specs=[
            pl.BlockSpec((RB, D), lambda i: (i, 0)),
            pl.BlockSpec((RB, D), lambda i: (i, 0)),
            pl.BlockSpec((RB, D), lambda i: (i, 0)),
            pl.BlockSpec((RB, D), lambda i: (i, 0)),
        ],
        out_specs=pl.BlockSpec((RB, D), lambda i: (i, 0)),
        out_shape=jax.ShapeDtypeStruct((N, D), jnp.float32),
    )(p0, p1, pd0, pd1)


def kernel(X, edge_index, Wq, Wk, Wphi, Wy, by, We, be):
    src = edge_index[0]
    dst = edge_index[1]
    Wqk = jnp.concatenate([Wq, Wk], axis=1)
    qk, vy = _node_stage(X, Wqk, Wy, by)
    g = _gather_stage(qk, src, dst)
    wve, wbb = _edge_stage(g, Wphi, We, be)
    p0, p1, pd0, pd1 = _scatter_stage(wve, wbb, vy, src, dst)
    return _final_stage(p0, p1, pd0, pd1)


# speedup vs baseline: 1.2376x; 1.0876x over previous
import functools

import jax
import jax.numpy as jnp
from jax import lax
from jax.experimental import pallas as pl
from jax.experimental.pallas import tpu as pltpu
from jax.experimental.pallas import tpu_sc as plsc

N = 10000
E = 320000
D = 128
A = 64

NCORES = 2
NSUB = 16
NTILES = NCORES * NSUB
EPT = E // NTILES
CH = 80
NCHUNK = EPT // CH
NPAD = 10240
RPT = NPAD // NSUB
ZR = 64

_MESH = plsc.VectorSubcoreMesh(core_axis_name="c", subcore_axis_name="s")


def _zv():
    return jnp.zeros((16,), jnp.float32)


def _node_body(x_ref, wqk_ref, wy_ref, by_ref, qk_ref, vy_ref):
    x = x_ref[...]
    qk_ref[...] = jnp.dot(x, wqk_ref[...], preferred_element_type=jnp.float32)
    vy_ref[...] = (
        jnp.dot(x, wy_ref[...], preferred_element_type=jnp.float32) + by_ref[...]
    )


def _node_stage(X, Wqk, Wy, by):
    return pl.pallas_call(
        _node_body,
        out_shape=[
            jax.ShapeDtypeStruct((N, D), jnp.float32),
            jax.ShapeDtypeStruct((N, D), jnp.float32),
        ],
    )(X, Wqk, Wy, by.reshape(1, D))


def _gather_body(qk_hbm, src_hbm, dst_hbm, g_hbm,
                 sidx, didx, qks, qkd, gb,
                 sem_si, sem_di, sem_gq, sem_gk, sem_wb):
    c = lax.axis_index("c")
    s = lax.axis_index("s")
    tile_base = (c * NSUB + s) * EPT

    def start_idx(i, b):
        base = tile_base + i * CH
        pltpu.async_copy(src_hbm.at[pl.ds(base, CH)], sidx.at[b], sem_si)
        pltpu.async_copy(dst_hbm.at[pl.ds(base, CH)], didx.at[b], sem_di)

    def wait_idx(b):
        pltpu.make_async_copy(src_hbm.at[pl.ds(0, CH)], sidx.at[b], sem_si).wait()
        pltpu.make_async_copy(dst_hbm.at[pl.ds(0, CH)], didx.at[b], sem_di).wait()

    def start_gathers(b):
        pltpu.async_copy(qk_hbm.at[sidx.at[b]], qks.at[b], sem_gq)
        pltpu.async_copy(qk_hbm.at[didx.at[b]], qkd.at[b], sem_gk)

    start_idx(0, 0)
    wait_idx(0)
    start_gathers(0)
    start_idx(1, 1)

    def chunk(i, carry):
        b = lax.rem(i, 2)
        nb = 1 - b

        @pl.when(i + 1 < NCHUNK)
        def _():
            wait_idx(nb)
            start_gathers(nb)

        pltpu.make_async_copy(qk_hbm.at[sidx.at[b]], qks.at[b], sem_gq).wait()
        pltpu.make_async_copy(qk_hbm.at[didx.at[b]], qkd.at[b], sem_gk).wait()

        @pl.when(i + 2 < NCHUNK)
        def _():
            start_idx(i + 2, b)

        @pl.when(i >= 1)
        def _():
            pltpu.make_async_copy(
                gb.at[nb], g_hbm.at[pl.ds(0, CH)], sem_wb
            ).wait()

        @plsc.parallel_loop(0, CH, unroll=8)
        def addrow(r):
            for l in range(A // 16):
                gb[b, r, pl.ds(l * 16, 16)] = (
                    qks[b, r, pl.ds(l * 16, 16)]
                    + qkd[b, r, pl.ds(A + l * 16, 16)]
                )
        base = tile_base + i * CH
        pltpu.async_copy(gb.at[b], g_hbm.at[pl.ds(base, CH)], sem_wb)
        return carry

    lax.fori_loop(0, NCHUNK, chunk, 0)
    pltpu.make_async_copy(gb.at[0], g_hbm.at[pl.ds(0, CH)], sem_wb).wait()


def _gather_stage(qk, src, dst):
    f = functools.partial(
        pl.kernel,
        out_type=jax.ShapeDtypeStruct((E, A), jnp.float32),
        mesh=_MESH,
        scratch_types=[
            pltpu.VMEM((2, CH), jnp.int32),
            pltpu.VMEM((2, CH), jnp.int32),
            pltpu.VMEM((2, CH, D), jnp.float32),
            pltpu.VMEM((2, CH, D), jnp.float32),
            pltpu.VMEM((2, CH, A), jnp.float32),
            pltpu.SemaphoreType.DMA,
            pltpu.SemaphoreType.DMA,
            pltpu.SemaphoreType.DMA,
            pltpu.SemaphoreType.DMA,
            pltpu.SemaphoreType.DMA,
        ],
    )(_gather_body)
    return f(qk, src, dst)


BE = 2560


def _edge_body(g_ref, wphi_ref, we_ref, be_ref, wve_ref, wb_ref):
    e = jnp.tanh(g_ref[...])
    sc = jnp.sum(e * wphi_ref[...], axis=1, keepdims=True)
    w = jnp.exp(sc)
    ve = jnp.dot(
        e.astype(jnp.bfloat16),
        we_ref[...].astype(jnp.bfloat16),
        preferred_element_type=jnp.float32,
    ) + be_ref[...]
    wve_ref[...] = w * ve
    wb_ref[...] = jnp.broadcast_to(w, (BE, 16))


def _edge_stage(g, Wphi, We, be):
    return pl.pallas_call(
        _edge_body,
        grid=(E // BE,),
        in_specs=[
            pl.BlockSpec((BE, A), lambda i: (i, 0)),
            pl.BlockSpec((1, A), lambda i: (0, 0)),
            pl.BlockSpec((A, D), lambda i: (0, 0)),
            pl.BlockSpec((1, D), lambda i: (0, 0)),
        ],
        out_specs=[
            pl.BlockSpec((BE, D), lambda i: (i, 0)),
            pl.BlockSpec((BE, 16), lambda i: (i, 0)),
        ],
        out_shape=[
            jax.ShapeDtypeStruct((E, D), jnp.float32),
            jax.ShapeDtypeStruct((E, 16), jnp.float32),
        ],
    )(g, Wphi.reshape(1, A), We, be.reshape(1, D))


def _scatter_body(wve_hbm, wbb_hbm, vy_hbm, src_hbm, dst_hbm, z_hbm,
                  p0_hbm, p1_hbm, pd0_hbm, pd1_hbm,
                  sidx, didx, mb, vyb, wbb,
                  sem_si, sem_di, sem_m, sem_w, sem_v, sem_sc, acc_sh):
    c = lax.axis_index("c")
    s = lax.axis_index("s")
    tile = c * NSUB + s
    tile_base = tile * EPT

    def zcopy(j, carry):
        pltpu.sync_copy(z_hbm, acc_sh.at[pl.ds(s * RPT + j * ZR, ZR)])
        return carry

    lax.fori_loop(0, RPT // ZR, zcopy, 0)
    plsc.subcore_barrier()

    def start_loads(i, b):
        base = tile_base + i * CH
        pltpu.async_copy(src_hbm.at[pl.ds(base, CH)], sidx.at[b], sem_si)
        pltpu.async_copy(dst_hbm.at[pl.ds(base, CH)], didx.at[b], sem_di)
        pltpu.async_copy(wve_hbm.at[pl.ds(base, CH)], mb.at[b], sem_m)
        pltpu.async_copy(wbb_hbm.at[pl.ds(base * 16, CH * 16)], wbb.at[b], sem_w)

    def wait_sc():
        pltpu.make_async_copy(mb.at[0], acc_sh.at[sidx.at[0]], sem_sc).wait()

    def wait_idx(b):
        pltpu.make_async_copy(src_hbm.at[pl.ds(0, CH)], sidx.at[b], sem_si).wait()
        pltpu.make_async_copy(dst_hbm.at[pl.ds(0, CH)], didx.at[b], sem_di).wait()

    start_loads(0, 0)
    wait_idx(0)
    pltpu.async_copy(vy_hbm.at[didx.at[0]], vyb.at[0], sem_v)

    def chunk(i, carry):
        b = lax.rem(i, 2)
        nb = 1 - b

        @pl.when(i >= 1)
        def _():
            wait_sc()

        @pl.when(i + 1 < NCHUNK)
        def _():
            start_loads(i + 1, nb)
            wait_idx(nb)
            pltpu.async_copy(vy_hbm.at[didx.at[nb]], vyb.at[nb], sem_v)

        pltpu.make_async_copy(wve_hbm.at[pl.ds(0, CH)], mb.at[b], sem_m).wait()
        pltpu.make_async_copy(wbb_hbm.at[pl.ds(0, CH * 16)], wbb.at[b], sem_w).wait()
        pltpu.make_async_copy(vy_hbm.at[didx.at[b]], vyb.at[b], sem_v).wait()

        @plsc.parallel_loop(0, CH, unroll=8)
        def frow(r):
            wv = wbb[b, pl.ds(r * 16, 16)]
            for l in range(D // 16):
                sl = pl.ds(l * 16, 16)
                mb[b, r, sl] = mb[b, r, sl] + wv * vyb[b, r, sl]
        pltpu.async_copy(mb.at[b], acc_sh.at[sidx.at[b]], sem_sc, add=True)
        return carry

    lax.fori_loop(0, NCHUNK, chunk, 0)
    wait_sc()
    plsc.subcore_barrier()

    row0 = s * RPT

    @pl.when(c == 0)
    def _():
        pltpu.sync_copy(acc_sh.at[pl.ds(row0, RPT)], p0_hbm.at[pl.ds(row0, RPT)])

    @pl.when(c == 1)
    def _():
        pltpu.sync_copy(acc_sh.at[pl.ds(row0, RPT)], p1_hbm.at[pl.ds(row0, RPT)])

    plsc.subcore_barrier()
    lax.fori_loop(0, RPT // ZR, zcopy, 0)
    plsc.subcore_barrier()

    def start_dloads(i, b):
        base = tile_base + i * CH
        pltpu.async_copy(src_hbm.at[pl.ds(base, CH)], sidx.at[b], sem_si)
        pltpu.async_copy(wbb_hbm.at[pl.ds(base * 16, CH * 16)], wbb.at[b], sem_w)

    start_dloads(0, 0)

    def dchunk(i, carry):
        b = lax.rem(i, 2)
        nb = 1 - b
        pltpu.make_async_copy(src_hbm.at[pl.ds(0, CH)], sidx.at[b], sem_si).wait()

        @pl.when(i >= 1)
        def _():
            wait_sc()

        @pl.when(i + 1 < NCHUNK)
        def _():
            start_dloads(i + 1, nb)

        pltpu.make_async_copy(wbb_hbm.at[pl.ds(0, CH * 16)], wbb.at[b], sem_w).wait()

        @plsc.parallel_loop(0, CH, unroll=8)
        def wrow(r):
            wv = wbb[b, pl.ds(r * 16, 16)]
            mb[b, r, pl.ds(0, 16)] = wv
        pltpu.async_copy(mb.at[b], acc_sh.at[sidx.at[b]], sem_sc, add=True)
        return carry

    lax.fori_loop(0, NCHUNK, dchunk, 0)
    wait_sc()
    plsc.subcore_barrier()

    @pl.when(c == 0)
    def _():
        pltpu.sync_copy(acc_sh.at[pl.ds(row0, RPT)], pd0_hbm.at[pl.ds(row0, RPT)])

    @pl.when(c == 1)
    def _():
        pltpu.sync_copy(acc_sh.at[pl.ds(row0, RPT)], pd1_hbm.at[pl.ds(row0, RPT)])


def _scatter_stage(wve, wbb, vy, src, dst):
    f = functools.partial(
        pl.kernel,
        out_type=[
            jax.ShapeDtypeStruct((NPAD, D), jnp.float32),
            jax.ShapeDtypeStruct((NPAD, D), jnp.float32),
            jax.ShapeDtypeStruct((NPAD, D), jnp.float32),
            jax.ShapeDtypeStruct((NPAD, D), jnp.float32),
        ],
        mesh=_MESH,
        scratch_types=[
            pltpu.VMEM((2, CH), jnp.int32),
            pltpu.VMEM((2, CH), jnp.int32),
            pltpu.VMEM((2, CH, D), jnp.float32),
            pltpu.VMEM((2, CH, D), jnp.float32),
            pltpu.VMEM((2, CH * 16), jnp.float32),
            pltpu.SemaphoreType.DMA,
            pltpu.SemaphoreType.DMA,
            pltpu.SemaphoreType.DMA,
            pltpu.SemaphoreType.DMA,
            pltpu.SemaphoreType.DMA,
            pltpu.SemaphoreType.DMA,
            pltpu.VMEM_SHARED((NPAD, D), jnp.float32),
        ],
    )(_scatter_body)
    return f(wve, wbb.reshape(E * 16), vy, src, dst, jnp.zeros((ZR, D), jnp.float32))


RB = 2000


def _final_body(p0_ref, p1_ref, pd0_ref, pd1_ref, o_ref):
    psum = p0_ref[...] + p1_ref[...]
    den = pd0_ref[...][:, 0:1] + pd1_ref[...][:, 0:1]
    o_ref[...] = psum / (den + 1e-9)


def _final_stage(p0, p1, pd0, pd1):
    return pl.pallas_call(
        _final_body,
        grid=(N // RB,),
        in_specs=[
            pl.BlockSpec((RB, D), lambda i: (i, 0)),
            pl.BlockSpec((RB, D), lambda i: (i, 0)),
            pl.BlockSpec((RB, D), lambda i: (i, 0)),
            pl.BlockSpec((RB, D), lambda i: (i, 0)),
        ],
        out_specs=pl.BlockSpec((RB, D), lambda i: (i, 0)),
        out_shape=jax.ShapeDtypeStruct((N, D), jnp.float32),
    )(p0, p1, pd0, pd1)


def kernel(X, edge_index, Wq, Wk, Wphi, Wy, by, We, be):
    src = edge_index[0]
    dst = edge_index[1]
    Wqk = jnp.concatenate([Wq, Wk], axis=1)
    qk, vy = _node_stage(X, Wqk, Wy, by)
    g = _gather_stage(qk, src, dst)
    wve, wbb = _edge_stage(g, Wphi, We, be)
    p0, p1, pd0, pd1 = _scatter_stage(wve, wbb, vy, src, dst)
    return _final_stage(p0, p1, pd0, pd1)
